# baseline (speedup 1.0000x reference)
import functools

import jax
import jax.numpy as jnp
from jax import lax
from jax.experimental import pallas as pl
from jax.experimental.pallas import tpu as pltpu
from jax.experimental.pallas import tpu_sc as plsc

EMBED = 32
PAD = 0
BATCH = 4096
SEQ = 50
VOCAB = 100000

NUM_CORES = 2
NUM_SUBCORES = 16
GROUPS = BATCH // 16
UNROLL = 4
OUTER = GROUPS // UNROLL
NBUF = 4
MAIN = (SEQ // NBUF) * NBUF


def _emb_body(src_idx, tgt_idx, src_tab, tgt_tab, src_out, tgt_out,
              row_v, ib0, ib1, ib2, ib3, ob0, ob1,
              sem_i0, sem_i1, sem_i2, sem_i3, sem_o0, sem_o1, sem_row):
    cid = lax.axis_index("c")
    sid = lax.axis_index("s")
    ibs = (ib0, ib1, ib2, ib3)
    sem_is = (sem_i0, sem_i1, sem_i2, sem_i3)
    obs = (ob0, ob1)
    sem_os = (sem_o0, sem_o1)

    def gather_row(ib, ob):
        @plsc.parallel_loop(0, BATCH, 16, unroll=UNROLL)
        def _(off):
            idx16 = ib[pl.ds(off, 16)]
            ob[pl.ds(off, 16)] = plsc.load_gather(row_v, [idx16])

    for tab, idxh, outh in ((src_tab, src_idx, src_out),
                            (tgt_tab, tgt_idx, tgt_out)):
        @pl.when(cid == (0 if tab is src_tab else 1))
        def _table():
            for phase in range(2):
                e = sid + phase * NUM_SUBCORES
                row_dma = pltpu.async_copy(tab.at[e], row_v, sem_row)
                for k in range(NBUF):
                    pltpu.async_copy(idxh.at[k], ibs[k], sem_is[k])
                row_dma.wait()
                head = row_v[pl.ds(0, 16)]
                row_v[pl.ds(0, 16)] = jnp.where(
                    lax.iota(jnp.int32, 16) == PAD, 0.0, head)

                def step(s, k):
                    ib, sem_i = ibs[k], sem_is[k]
                    ob, sem_o = obs[k % 2], sem_os[k % 2]
                    pltpu.make_async_copy(idxh.at[0], ib, sem_i).wait()

                    @pl.when(s >= 2)
                    def _():
                        pltpu.make_async_copy(ob, outh.at[0, e], sem_o).wait()

                    gather_row(ib, ob)

                    @pl.when(s + NBUF < SEQ)
                    def _():
                        pltpu.async_copy(idxh.at[s + NBUF], ib, sem_i)

                    pltpu.async_copy(ob, outh.at[s, e], sem_o)

                def quad(i, _):
                    for k in range(NBUF):
                        step(NBUF * i + k, k)
                    return 0

                lax.fori_loop(0, MAIN // NBUF, quad, 0)
                for s in range(MAIN, SEQ):
                    step(s, s % NBUF)
                pltpu.make_async_copy(ob0, outh.at[0, e], sem_o0).wait()
                pltpu.make_async_copy(ob1, outh.at[0, e], sem_o1).wait()


_emb_kernel = functools.partial(
    pl.kernel,
    mesh=plsc.VectorSubcoreMesh(core_axis_name="c", subcore_axis_name="s"),
    out_type=(
        jax.ShapeDtypeStruct((SEQ, EMBED, BATCH), jnp.float32),
        jax.ShapeDtypeStruct((SEQ, EMBED, BATCH), jnp.float32),
    ),
    scratch_types=[
        pltpu.VMEM((VOCAB,), jnp.float32),
        pltpu.VMEM((BATCH,), jnp.int32),
        pltpu.VMEM((BATCH,), jnp.int32),
        pltpu.VMEM((BATCH,), jnp.int32),
        pltpu.VMEM((BATCH,), jnp.int32),
        pltpu.VMEM((BATCH,), jnp.float32),
        pltpu.VMEM((BATCH,), jnp.float32),
        pltpu.SemaphoreType.DMA,
        pltpu.SemaphoreType.DMA,
        pltpu.SemaphoreType.DMA,
        pltpu.SemaphoreType.DMA,
        pltpu.SemaphoreType.DMA,
        pltpu.SemaphoreType.DMA,
        pltpu.SemaphoreType.DMA,
    ],
    compiler_params=pltpu.CompilerParams(
        use_tc_tiling_on_sc=True, needs_layout_passes=False),
)(_emb_body)


@jax.jit
def kernel(src_indices, tgt_indices, src_table, tgt_table):
    si = src_indices.T.astype(jnp.int32)
    ti = tgt_indices.T.astype(jnp.int32)
    st = src_table.T
    tt = tgt_table.T
    src_out, tgt_out = _emb_kernel(si, ti, st, tt)
    return (jnp.transpose(src_out, (2, 0, 1)),
            jnp.transpose(tgt_out, (2, 0, 1)))


# speedup vs baseline: 1.0080x; 1.0080x over previous
import functools

import jax
import jax.numpy as jnp
from jax import lax
from jax.experimental import pallas as pl
from jax.experimental.pallas import tpu as pltpu
from jax.experimental.pallas import tpu_sc as plsc

EMBED = 32
PAD = 0
BATCH = 4096
SEQ = 50
VOCAB = 100000

NUM_CORES = 2
NUM_SUBCORES = 16
UNROLL = 8
NBUF = 4
MAIN = (SEQ // NBUF) * NBUF


def _emb_body(src_idx, tgt_idx, src_tab, tgt_tab, src_out, tgt_out,
              row_v, ib0, ib1, ib2, ib3, ob0, ob1,
              sem_i0, sem_i1, sem_i2, sem_i3, sem_o0, sem_o1, sem_row):
    cid = lax.axis_index("c")
    sid = lax.axis_index("s")
    ibs = (ib0, ib1, ib2, ib3)
    sem_is = (sem_i0, sem_i1, sem_i2, sem_i3)
    obs = (ob0, ob1)
    sem_os = (sem_o0, sem_o1)

    def gather_row(ib, ob):
        @plsc.parallel_loop(0, BATCH, 16, unroll=UNROLL)
        def _(off):
            idx16 = ib[pl.ds(off, 16)]
            ob[pl.ds(off, 16)] = plsc.load_gather(row_v, [idx16])

    for tab, idxh, outh in ((src_tab, src_idx, src_out),
                            (tgt_tab, tgt_idx, tgt_out)):
        @pl.when(cid == (0 if tab is src_tab else 1))
        def _table():
            for phase in range(2):
                e = sid + phase * NUM_SUBCORES
                row_dma = pltpu.async_copy(tab.at[e], row_v, sem_row)
                for k in range(NBUF):
                    pltpu.async_copy(idxh.at[k], ibs[k], sem_is[k])
                row_dma.wait()
                head = row_v[pl.ds(0, 16)]
                row_v[pl.ds(0, 16)] = jnp.where(
                    lax.iota(jnp.int32, 16) == PAD, 0.0, head)

                def step(s, k):
                    ib, sem_i = ibs[k], sem_is[k]
                    ob, sem_o = obs[k % 2], sem_os[k % 2]
                    pltpu.make_async_copy(idxh.at[0], ib, sem_i).wait()

                    @pl.when(s >= 2)
                    def _():
                        pltpu.make_async_copy(ob, outh.at[0, e], sem_o).wait()

                    gather_row(ib, ob)

                    @pl.when(s + NBUF < SEQ)
                    def _():
                        pltpu.async_copy(idxh.at[s + NBUF], ib, sem_i)

                    pltpu.async_copy(ob, outh.at[s, e], sem_o)

                def quad(i, _):
                    for k in range(NBUF):
                        step(NBUF * i + k, k)
                    return 0

                lax.fori_loop(0, MAIN // NBUF, quad, 0)
                for s in range(MAIN, SEQ):
                    step(s, s % NBUF)
                pltpu.make_async_copy(ob0, outh.at[0, e], sem_o0).wait()
                pltpu.make_async_copy(ob1, outh.at[0, e], sem_o1).wait()


_emb_kernel = functools.partial(
    pl.kernel,
    mesh=plsc.VectorSubcoreMesh(core_axis_name="c", subcore_axis_name="s"),
    out_type=(
        jax.ShapeDtypeStruct((SEQ, EMBED, BATCH), jnp.float32),
        jax.ShapeDtypeStruct((SEQ, EMBED, BATCH), jnp.float32),
    ),
    scratch_types=[
        pltpu.VMEM((VOCAB,), jnp.float32),
        pltpu.VMEM((BATCH,), jnp.int32),
        pltpu.VMEM((BATCH,), jnp.int32),
        pltpu.VMEM((BATCH,), jnp.int32),
        pltpu.VMEM((BATCH,), jnp.int32),
        pltpu.VMEM((BATCH,), jnp.float32),
        pltpu.VMEM((BATCH,), jnp.float32),
        pltpu.SemaphoreType.DMA,
        pltpu.SemaphoreType.DMA,
        pltpu.SemaphoreType.DMA,
        pltpu.SemaphoreType.DMA,
        pltpu.SemaphoreType.DMA,
        pltpu.SemaphoreType.DMA,
        pltpu.SemaphoreType.DMA,
    ],
    compiler_params=pltpu.CompilerParams(
        use_tc_tiling_on_sc=True, needs_layout_passes=False),
)(_emb_body)


@jax.jit
def kernel(src_indices, tgt_indices, src_table, tgt_table):
    si = src_indices.T.astype(jnp.int32)
    ti = tgt_indices.T.astype(jnp.int32)
    st = src_table.T
    tt = tgt_table.T
    src_out, tgt_out = _emb_kernel(si, ti, st, tt)
    return (jnp.transpose(src_out, (2, 0, 1)),
            jnp.transpose(tgt_out, (2, 0, 1)))
